# baseline (speedup 1.0000x reference)
import functools
import math

import jax
import jax.numpy as jnp
from jax import lax
from jax.experimental import pallas as pl
from jax.experimental.pallas import tpu as pltpu
from jax.experimental.pallas import tpu_sc as plsc

NUM_BANDS = 32
NORMALIZE_SCALE = 6.87
FDIM = 2 * (1 + 2 * NUM_BANDS)

_TWO_PI = 2.0 * math.pi
_RND = 1.5 * 2.0**23


def _fast_sin(angle):
    n = (angle * (1.0 / _TWO_PI) + _RND) - _RND
    t = angle - n * _TWO_PI
    s = t * t
    p = 2.17325696e-06
    p = p * s + -1.93162699e-04
    p = p * s + 8.31238828e-03
    p = p * s + -1.66632594e-01
    p = p * s + 9.99984593e-01
    return p * t


def _sc_knn(posx, posy, kk):
    B, L = posx.shape
    info = plsc.get_sparse_core_info()
    NC, NS = info.num_cores, info.num_subcores
    NW = NC * NS
    rows_w = (B * L) // NW
    wpb = L // rows_w
    nchunks = L // 16
    mesh = plsc.VectorSubcoreMesh(core_axis_name="c", subcore_axis_name="s")

    @functools.partial(
        pl.kernel,
        mesh=mesh,
        compiler_params=pltpu.CompilerParams(needs_layout_passes=False),
        out_type=[
            jax.ShapeDtypeStruct((NW, rows_w * kk), jnp.int32),
            jax.ShapeDtypeStruct((NW, rows_w * kk), jnp.float32),
            jax.ShapeDtypeStruct((NW, rows_w * kk), jnp.float32),
        ],
        scratch_types=[
            pltpu.VMEM((L,), jnp.float32),
            pltpu.VMEM((L,), jnp.float32),
            pltpu.VMEM((rows_w * kk,), jnp.int32),
            pltpu.VMEM((rows_w * kk,), jnp.float32),
            pltpu.VMEM((rows_w * kk,), jnp.float32),
        ],
    )
    def knn(posx_hbm, posy_hbm, idx_hbm, dx_hbm, dy_hbm, px, py, ib, xb, yb):
        wid = lax.axis_index("s") * NC + lax.axis_index("c")
        batch = wid // wpb
        base = (wid % wpb) * rows_w
        pltpu.sync_copy(posx_hbm.at[batch], px)
        pltpu.sync_copy(posy_hbm.at[batch], py)
        lane = lax.broadcasted_iota(jnp.int32, (16,), 0)
        inf = jnp.float32(jnp.inf)

        def row_body(r, carry):
            q = base + r
            qv = jnp.full((16,), q, jnp.int32)
            xq = plsc.load_gather(px, [qv])
            yq = plsc.load_gather(py, [qv])
            plsc.store_scatter(px, [qv], jnp.full((16,), inf))

            def chunk_body(c, st):
                bd, bi, thr = st
                off = pl.multiple_of(c * 16, 16)
                xj = px[pl.ds(off, 16)]
                yj = py[pl.ds(off, 16)]
                dx = xj - xq
                dy = yj - yq
                d = dx * dx + dy * dy
                cnt = plsc.all_reduce_population_count(d < thr)

                def merge(st2):
                    bd0, bi0, _ = st2
                    ci = c * 16 + lane
                    dd, di = plsc.sort_key_val(d, ci, descending=True)
                    take = dd < bd0
                    nd = jnp.where(take, dd, bd0)
                    ni = jnp.where(take, di, bi0)
                    bd1, bi1 = plsc.sort_key_val(nd, ni)
                    return bd1, bi1, jnp.full((16,), bd1[15])

                return lax.cond(cnt[0] > 0, merge, lambda s: s, (bd, bi, thr))

            init = (jnp.full((16,), inf), jnp.full((16,), L, jnp.int32),
                    jnp.full((16,), inf))
            bd, bi, _ = lax.fori_loop(0, nchunks, chunk_body, init)
            plsc.store_scatter(px, [qv], xq)
            nx = plsc.load_gather(px, [bi])
            ny = plsc.load_gather(py, [bi])
            o = pl.multiple_of(r * kk, kk)
            ib[pl.ds(o, kk)] = bi
            xb[pl.ds(o, kk)] = nx - xq
            yb[pl.ds(o, kk)] = ny - yq
            return carry

        lax.fori_loop(0, rows_w, row_body, 0)
        pltpu.sync_copy(ib, idx_hbm.at[wid])
        pltpu.sync_copy(xb, dx_hbm.at[wid])
        pltpu.sync_copy(yb, dy_hbm.at[wid])

    return knn(posx, posy)


def _encode_body(dx_ref, dy_ref, rpe_ref, dist_ref, self_ref, *, rb, srb):
    dx = dx_ref[...]
    dy = dy_ref[...]
    dist_ref[...] = jnp.sqrt(dx * dx + dy * dy + 1e-8)
    f = jax.lax.broadcasted_iota(jnp.int32, (1, FDIM), 1)
    g = f % 65
    isy = f >= 65
    iscos = g >= 33
    israw = g == 0
    freq = jnp.where(iscos, g - 32, g).astype(jnp.float32)
    phase = jnp.where(iscos, 0.5 * math.pi, 0.0)
    dxc = dx * (1.0 / NORMALIZE_SCALE)
    dxc = dxc / (1.0 + jnp.abs(dxc))
    dyc = dy * (1.0 / NORMALIZE_SCALE)
    dyc = dyc / (1.0 + jnp.abs(dyc))
    vc = jnp.where(isy, dyc, dxc)
    enc = _fast_sin(vc * (freq * math.pi) + phase)
    rpe_ref[...] = jnp.where(israw, vc, enc)
    col = jax.lax.broadcasted_iota(jnp.int32, (srb, FDIM), 1)
    self_ref[...] = jnp.where((col % 65) >= 33, 1.0, 0.0)


def kernel(positions, k):
    B, L, _ = positions.shape
    kk = min(16, L - 1)
    posx = positions[..., 0]
    posy = positions[..., 1]

    idx, dxs, dys = _sc_knn(posx, posy, kk)
    idx = idx.reshape(B, L, kk)

    N = B * L * kk
    NS = B * L
    RB2 = 1024
    grid2 = (N // RB2,)
    SRB = NS // (N // RB2)
    v_spec = pl.BlockSpec((RB2, 1), lambda i: (i, 0))
    rpe, dist, self_rpe = pl.pallas_call(
        functools.partial(_encode_body, rb=RB2, srb=SRB),
        grid=grid2,
        in_specs=[v_spec, v_spec],
        out_specs=[
            pl.BlockSpec((RB2, FDIM), lambda i: (i, 0)),
            v_spec,
            pl.BlockSpec((SRB, FDIM), lambda i: (i, 0)),
        ],
        out_shape=[
            jax.ShapeDtypeStruct((N, FDIM), jnp.float32),
            jax.ShapeDtypeStruct((N, 1), jnp.float32),
            jax.ShapeDtypeStruct((NS, FDIM), jnp.float32),
        ],
    )(dxs.reshape(N, 1), dys.reshape(N, 1))

    topk_indices = idx + jnp.asarray(k - kk, dtype=idx.dtype)
    return (
        topk_indices,
        rpe.reshape(B, L, kk, FDIM),
        self_rpe.reshape(B, L, 1, FDIM),
        dist.reshape(B, L, kk),
    )
